# baseline (speedup 1.0000x reference)
import functools

import jax
import jax.numpy as jnp
from jax import lax
from jax.experimental import pallas as pl
from jax.experimental.pallas import tpu as pltpu
from jax.experimental.pallas import tpu_sc as plsc


def _add_block(x_ref, pos_ref, o_ref):
    o_ref[...] = x_ref[...] + pos_ref[...]


def _kernel_tc(X, pos_table):
    B, S, D = X.shape
    bs = 2048
    grid = (S // bs, B)
    out = pl.pallas_call(
        _add_block,
        grid=grid,
        in_specs=[
            pl.BlockSpec((1, bs, D), lambda s, b: (b, s, 0)),
            pl.BlockSpec((bs, D), lambda s, b: (s, 0)),
        ],
        out_specs=pl.BlockSpec((1, bs, D), lambda s, b: (b, s, 0)),
        out_shape=jax.ShapeDtypeStruct((B, S, D), X.dtype),
    )(X, pos_table[:S])
    return out


_NC, _NS, _NL = 2, 16, 16
_NW = _NC * _NS
_CHUNK = 32768


def _sc_body(x_hbm, pos_hbm, o_hbm, xb, pb, sem_x, sem_p):
    w = lax.axis_index("c") * _NS + lax.axis_index("s")
    span = x_hbm.shape[0] // _NW
    pos_total = pos_hbm.shape[0]
    x0 = w * span
    p0 = (w * span) % pos_total

    def chunk(i, carry):
        off = i * _CHUNK
        cx = pltpu.make_async_copy(x_hbm.at[pl.ds(x0 + off, _CHUNK)], xb, sem_x)
        cp = pltpu.make_async_copy(pos_hbm.at[pl.ds(p0 + off, _CHUNK)], pb, sem_p)
        cx.start()
        cp.start()
        cx.wait()
        cp.wait()

        def add(v, c):
            sl = pl.ds(v * _NL, _NL)
            xb[sl] = xb[sl] + pb[sl]
            return c

        lax.fori_loop(0, _CHUNK // _NL, add, 0, unroll=8)
        pltpu.sync_copy(xb, o_hbm.at[pl.ds(x0 + off, _CHUNK)])
        return carry

    lax.fori_loop(0, span // _CHUNK, chunk, 0)


def _kernel_sc(X, pos_table):
    B, S, D = X.shape
    n = B * S * D
    mesh = plsc.VectorSubcoreMesh(core_axis_name="c", subcore_axis_name="s")
    k = functools.partial(
        pl.kernel,
        mesh=mesh,
        out_type=jax.ShapeDtypeStruct((n,), jnp.float32),
        scratch_types=[
            pltpu.VMEM((_CHUNK,), jnp.float32),
            pltpu.VMEM((_CHUNK,), jnp.float32),
            pltpu.SemaphoreType.DMA,
            pltpu.SemaphoreType.DMA,
        ],
    )(_sc_body)
    out = k(X.reshape(n), pos_table[:S].reshape(S * D))
    return out.reshape(B, S, D)


kernel = _kernel_sc


# speedup vs baseline: 1.0868x; 1.0868x over previous
import functools

import jax
import jax.numpy as jnp
from jax import lax
from jax.experimental import pallas as pl
from jax.experimental.pallas import tpu as pltpu
from jax.experimental.pallas import tpu_sc as plsc


def _add_block(x_ref, pos_ref, o_ref):
    o_ref[...] = x_ref[...] + pos_ref[...]


def _kernel_tc(X, pos_table):
    B, S, D = X.shape
    bs = 2048
    grid = (S // bs, B)
    out = pl.pallas_call(
        _add_block,
        grid=grid,
        in_specs=[
            pl.BlockSpec((1, bs, D), lambda s, b: (b, s, 0)),
            pl.BlockSpec((bs, D), lambda s, b: (s, 0)),
        ],
        out_specs=pl.BlockSpec((1, bs, D), lambda s, b: (b, s, 0)),
        out_shape=jax.ShapeDtypeStruct((B, S, D), X.dtype),
    )(X, pos_table[:S])
    return out


_NC, _NS, _NL = 2, 16, 16
_NW = _NC * _NS
_CHUNK = 16384


def _sc_body(x_hbm, pos_hbm, o_hbm,
             xb0, pb0, xb1, pb1,
             sx0, sp0, sx1, sp1, so0, so1):
    w = lax.axis_index("c") * _NS + lax.axis_index("s")
    span = x_hbm.shape[0] // _NW
    pos_total = pos_hbm.shape[0]
    x0 = w * span
    p0 = (w * span) % pos_total
    nchunks = span // _CHUNK

    def load(g, xb, pb, sx, sp):
        off = g * _CHUNK
        pltpu.make_async_copy(x_hbm.at[pl.ds(x0 + off, _CHUNK)], xb, sx).start()
        pltpu.make_async_copy(pos_hbm.at[pl.ds(p0 + off, _CHUNK)], pb, sp).start()

    def wait_load(xb, pb, sx, sp):
        pltpu.make_async_copy(x_hbm.at[pl.ds(x0, _CHUNK)], xb, sx).wait()
        pltpu.make_async_copy(pos_hbm.at[pl.ds(p0, _CHUNK)], pb, sp).wait()

    def compute(xb, pb):
        def add(v, c):
            sl = pl.ds(v * _NL, _NL)
            xb[sl] = xb[sl] + pb[sl]
            return c
        lax.fori_loop(0, _CHUNK // _NL, add, 0, unroll=8)

    def store(g, xb, so):
        off = g * _CHUNK
        pltpu.make_async_copy(xb, o_hbm.at[pl.ds(x0 + off, _CHUNK)], so).start()

    def wait_store(xb, so):
        pltpu.make_async_copy(xb, o_hbm.at[pl.ds(x0, _CHUNK)], so).wait()

    load(0, xb0, pb0, sx0, sp0)
    load(1, xb1, pb1, sx1, sp1)

    def body(i, carry):
        g0 = i * 2
        g2 = lax.min(g0 + 2, nchunks - 2)
        g3 = lax.min(g0 + 3, nchunks - 1)

        wait_load(xb0, pb0, sx0, sp0)
        compute(xb0, pb0)
        store(g0, xb0, so0)

        wait_load(xb1, pb1, sx1, sp1)
        compute(xb1, pb1)
        store(g0 + 1, xb1, so1)

        wait_store(xb0, so0)
        load(g2, xb0, pb0, sx0, sp0)
        wait_store(xb1, so1)
        load(g3, xb1, pb1, sx1, sp1)
        return carry

    lax.fori_loop(0, nchunks // 2, body, 0)
    wait_load(xb0, pb0, sx0, sp0)
    wait_load(xb1, pb1, sx1, sp1)


def _kernel_sc(X, pos_table):
    B, S, D = X.shape
    n = B * S * D
    mesh = plsc.VectorSubcoreMesh(core_axis_name="c", subcore_axis_name="s")
    k = functools.partial(
        pl.kernel,
        mesh=mesh,
        out_type=jax.ShapeDtypeStruct((n,), jnp.float32),
        scratch_types=[
            pltpu.VMEM((_CHUNK,), jnp.float32),
            pltpu.VMEM((_CHUNK,), jnp.float32),
            pltpu.VMEM((_CHUNK,), jnp.float32),
            pltpu.VMEM((_CHUNK,), jnp.float32),
            pltpu.SemaphoreType.DMA,
            pltpu.SemaphoreType.DMA,
            pltpu.SemaphoreType.DMA,
            pltpu.SemaphoreType.DMA,
            pltpu.SemaphoreType.DMA,
            pltpu.SemaphoreType.DMA,
        ],
    )(_sc_body)
    out = k(X.reshape(n), pos_table[:S].reshape(S * D))
    return out.reshape(B, S, D)


kernel = _kernel_sc


# speedup vs baseline: 1.6251x; 1.4953x over previous
import functools

import jax
import jax.numpy as jnp
from jax import lax
from jax.experimental import pallas as pl
from jax.experimental.pallas import tpu as pltpu
from jax.experimental.pallas import tpu_sc as plsc


def _add_block(x_ref, pos_ref, o_ref):
    o_ref[...] = x_ref[...] + pos_ref[...]


def _kernel_tc(X, pos_table):
    B, S, D = X.shape
    bs = 2048
    grid = (S // bs, B)
    out = pl.pallas_call(
        _add_block,
        grid=grid,
        in_specs=[
            pl.BlockSpec((1, bs, D), lambda s, b: (b, s, 0)),
            pl.BlockSpec((bs, D), lambda s, b: (s, 0)),
        ],
        out_specs=pl.BlockSpec((1, bs, D), lambda s, b: (b, s, 0)),
        out_shape=jax.ShapeDtypeStruct((B, S, D), X.dtype),
    )(X, pos_table[:S])
    return out


_NC, _NS, _NL = 2, 16, 16
_NW = _NC * _NS
_CHUNK = 16384


def _sc_body(x_hbm, pos_hbm, o_hbm,
             xb0, pb0, xb1, pb1,
             sx0, sp0, sx1, sp1, so0, so1):
    w = lax.axis_index("c") * _NS + lax.axis_index("s")
    span = x_hbm.shape[0] // _NW
    pos_total = pos_hbm.shape[0]
    x0 = w * span
    p0 = (w * span) % pos_total
    nchunks = span // _CHUNK

    def load(g, xb, pb, sx, sp):
        off = g * _CHUNK
        pltpu.make_async_copy(x_hbm.at[pl.ds(x0 + off, _CHUNK)], xb, sx).start()
        pltpu.make_async_copy(pos_hbm.at[pl.ds(p0 + off, _CHUNK)], pb, sp).start()

    def wait_load(xb, pb, sx, sp):
        pltpu.make_async_copy(x_hbm.at[pl.ds(x0, _CHUNK)], xb, sx).wait()
        pltpu.make_async_copy(pos_hbm.at[pl.ds(p0, _CHUNK)], pb, sp).wait()

    def compute(xb, pb):
        @plsc.parallel_loop(0, _CHUNK, step=_NL, unroll=8)
        def add(v):
            sl = pl.ds(v, _NL)
            xb[sl] = xb[sl] + pb[sl]

    def store(g, xb, so):
        off = g * _CHUNK
        pltpu.make_async_copy(xb, o_hbm.at[pl.ds(x0 + off, _CHUNK)], so).start()

    def wait_store(xb, so):
        pltpu.make_async_copy(xb, o_hbm.at[pl.ds(x0, _CHUNK)], so).wait()

    load(0, xb0, pb0, sx0, sp0)
    load(1, xb1, pb1, sx1, sp1)

    def body(i, carry):
        g0 = i * 2
        g2 = lax.min(g0 + 2, nchunks - 2)
        g3 = lax.min(g0 + 3, nchunks - 1)

        wait_load(xb0, pb0, sx0, sp0)
        compute(xb0, pb0)
        store(g0, xb0, so0)

        wait_load(xb1, pb1, sx1, sp1)
        compute(xb1, pb1)
        store(g0 + 1, xb1, so1)

        wait_store(xb0, so0)
        load(g2, xb0, pb0, sx0, sp0)
        wait_store(xb1, so1)
        load(g3, xb1, pb1, sx1, sp1)
        return carry

    lax.fori_loop(0, nchunks // 2, body, 0)
    wait_load(xb0, pb0, sx0, sp0)
    wait_load(xb1, pb1, sx1, sp1)


def _kernel_sc(X, pos_table):
    B, S, D = X.shape
    n = B * S * D
    mesh = plsc.VectorSubcoreMesh(core_axis_name="c", subcore_axis_name="s")
    k = functools.partial(
        pl.kernel,
        mesh=mesh,
        out_type=jax.ShapeDtypeStruct((n,), jnp.float32),
        scratch_types=[
            pltpu.VMEM((_CHUNK,), jnp.float32),
            pltpu.VMEM((_CHUNK,), jnp.float32),
            pltpu.VMEM((_CHUNK,), jnp.float32),
            pltpu.VMEM((_CHUNK,), jnp.float32),
            pltpu.SemaphoreType.DMA,
            pltpu.SemaphoreType.DMA,
            pltpu.SemaphoreType.DMA,
            pltpu.SemaphoreType.DMA,
            pltpu.SemaphoreType.DMA,
            pltpu.SemaphoreType.DMA,
        ],
    )(_sc_body)
    out = k(X.reshape(n), pos_table[:S].reshape(S * D))
    return out.reshape(B, S, D)


kernel = _kernel_sc


# speedup vs baseline: 3.8262x; 2.3544x over previous
import functools

import jax
import jax.numpy as jnp
from jax import lax
from jax.experimental import pallas as pl
from jax.experimental.pallas import tpu as pltpu
from jax.experimental.pallas import tpu_sc as plsc


def _add_block(x_ref, pos_ref, o_ref):
    o_ref[...] = x_ref[...] + pos_ref[...]


def _kernel_tc(X, pos_table):
    B, S, D = X.shape
    bs = 2048
    grid = (S // bs, B)
    out = pl.pallas_call(
        _add_block,
        grid=grid,
        in_specs=[
            pl.BlockSpec((1, bs, D), lambda s, b: (b, s, 0)),
            pl.BlockSpec((bs, D), lambda s, b: (s, 0)),
        ],
        out_specs=pl.BlockSpec((1, bs, D), lambda s, b: (b, s, 0)),
        out_shape=jax.ShapeDtypeStruct((B, S, D), X.dtype),
    )(X, pos_table[:S])
    return out


_NC, _NS, _NL = 2, 16, 16
_NW = _NC * _NS
_CR = 16


def _sc_body(x_hbm, pos_hbm, o_hbm,
             xb0, pb0, xb1, pb1,
             sx0, sp0, sx1, sp1, so0, so1):
    w = lax.axis_index("c") * _NS + lax.axis_index("s")
    B, S, D = x_hbm.shape
    rows = (B * S) // _NW
    wpb = S // rows
    b = w // wpb
    r0 = (w % wpb) * rows
    nchunks = rows // _CR

    def load(g, xb, pb, sx, sp):
        r = r0 + g * _CR
        pltpu.make_async_copy(x_hbm.at[b, pl.ds(r, _CR), :], xb, sx).start()
        pltpu.make_async_copy(pos_hbm.at[pl.ds(r, _CR), :], pb, sp).start()

    def wait_load(xb, pb, sx, sp):
        pltpu.make_async_copy(x_hbm.at[b, pl.ds(r0, _CR), :], xb, sx).wait()
        pltpu.make_async_copy(pos_hbm.at[pl.ds(r0, _CR), :], pb, sp).wait()

    def compute(xb, pb):
        @plsc.parallel_loop(0, _CR * D, step=_NL, unroll=8)
        def add(v):
            r = v // D
            sl = pl.ds(v % D, _NL)
            xb[r, sl] = xb[r, sl] + pb[r, sl]

    def store(g, xb, so):
        r = r0 + g * _CR
        pltpu.make_async_copy(xb, o_hbm.at[b, pl.ds(r, _CR), :], so).start()

    def wait_store(xb, so):
        pltpu.make_async_copy(xb, o_hbm.at[b, pl.ds(r0, _CR), :], so).wait()

    load(0, xb0, pb0, sx0, sp0)
    load(1, xb1, pb1, sx1, sp1)

    def body(i, carry):
        g0 = i * 2
        g2 = lax.min(g0 + 2, nchunks - 2)
        g3 = lax.min(g0 + 3, nchunks - 1)

        wait_load(xb0, pb0, sx0, sp0)
        compute(xb0, pb0)
        store(g0, xb0, so0)

        wait_load(xb1, pb1, sx1, sp1)
        compute(xb1, pb1)
        store(g0 + 1, xb1, so1)

        wait_store(xb0, so0)
        load(g2, xb0, pb0, sx0, sp0)
        wait_store(xb1, so1)
        load(g3, xb1, pb1, sx1, sp1)
        return carry

    lax.fori_loop(0, nchunks // 2, body, 0)
    wait_load(xb0, pb0, sx0, sp0)
    wait_load(xb1, pb1, sx1, sp1)


def _kernel_sc(X, pos_table):
    B, S, D = X.shape
    mesh = plsc.VectorSubcoreMesh(core_axis_name="c", subcore_axis_name="s")
    k = functools.partial(
        pl.kernel,
        mesh=mesh,
        out_type=jax.ShapeDtypeStruct((B, S, D), jnp.float32),
        scratch_types=[
            pltpu.VMEM((_CR, D), jnp.float32),
            pltpu.VMEM((_CR, D), jnp.float32),
            pltpu.VMEM((_CR, D), jnp.float32),
            pltpu.VMEM((_CR, D), jnp.float32),
            pltpu.SemaphoreType.DMA,
            pltpu.SemaphoreType.DMA,
            pltpu.SemaphoreType.DMA,
            pltpu.SemaphoreType.DMA,
            pltpu.SemaphoreType.DMA,
            pltpu.SemaphoreType.DMA,
        ],
    )(_sc_body)
    return k(X, pos_table[:S])


kernel = _kernel_sc


# speedup vs baseline: 7.8645x; 2.0554x over previous
import functools

import jax
import jax.numpy as jnp
from jax import lax
from jax.experimental import pallas as pl
from jax.experimental.pallas import tpu as pltpu
from jax.experimental.pallas import tpu_sc as plsc


def _add_block(x_ref, pos_ref, o_ref):
    o_ref[...] = x_ref[...] + pos_ref[...]


def _kernel_tc(X, pos_table):
    B, S, D = X.shape
    bs = 2048
    grid = (S // bs, B)
    out = pl.pallas_call(
        _add_block,
        grid=grid,
        in_specs=[
            pl.BlockSpec((1, bs, D), lambda s, b: (b, s, 0)),
            pl.BlockSpec((bs, D), lambda s, b: (s, 0)),
        ],
        out_specs=pl.BlockSpec((1, bs, D), lambda s, b: (b, s, 0)),
        out_shape=jax.ShapeDtypeStruct((B, S, D), X.dtype),
    )(X, pos_table[:S])
    return out


_NC, _NS, _NL = 2, 16, 16
_NW = _NC * _NS
_CR = 16


def _sc_body(x_hbm, pos_hbm, o_hbm,
             xb0, pb0, xb1, pb1,
             sx0, sp0, sx1, sp1, so0, so1):
    w = lax.axis_index("c") * _NS + lax.axis_index("s")
    B, S, D = x_hbm.shape
    rows = (B * S) // _NW
    wpb = S // rows
    b = w // wpb
    r0 = (w % wpb) * rows
    nchunks = rows // _CR

    def load(g, xb, pb, sx, sp):
        r = r0 + g * _CR
        pltpu.make_async_copy(x_hbm.at[b, pl.ds(r, _CR), :], xb, sx).start()
        pltpu.make_async_copy(pos_hbm.at[pl.ds(r, _CR), :], pb, sp).start()

    def wait_load(xb, pb, sx, sp):
        pltpu.make_async_copy(x_hbm.at[b, pl.ds(r0, _CR), :], xb, sx).wait()
        pltpu.make_async_copy(pos_hbm.at[pl.ds(r0, _CR), :], pb, sp).wait()

    def compute(xb, pb):
        @plsc.parallel_loop(0, _CR * D, step=_NL, unroll=8)
        def add(v):
            r = v // D
            sl = pl.ds(v % D, _NL)
            xb[r, sl] = xb[r, sl] + pb[r, sl]

    def store(g, xb, so):
        r = r0 + g * _CR
        pltpu.make_async_copy(xb, o_hbm.at[b, pl.ds(r, _CR), :], so).start()

    def wait_store(xb, so):
        pltpu.make_async_copy(xb, o_hbm.at[b, pl.ds(r0, _CR), :], so).wait()

    load(0, xb0, pb0, sx0, sp0)
    load(1, xb1, pb1, sx1, sp1)

    def body(i, carry):
        g0 = i * 2
        g2 = lax.min(g0 + 2, nchunks - 2)
        g3 = lax.min(g0 + 3, nchunks - 1)

        wait_load(xb0, pb0, sx0, sp0)
        compute(xb0, pb0)
        store(g0, xb0, so0)

        wait_load(xb1, pb1, sx1, sp1)
        compute(xb1, pb1)
        store(g0 + 1, xb1, so1)

        wait_store(xb0, so0)
        load(g2, xb0, pb0, sx0, sp0)
        wait_store(xb1, so1)
        load(g3, xb1, pb1, sx1, sp1)
        return carry

    lax.fori_loop(0, nchunks // 2, body, 0)
    wait_load(xb0, pb0, sx0, sp0)
    wait_load(xb1, pb1, sx1, sp1)


def _kernel_sc(X, pos_table):
    B, S, D = X.shape
    mesh = plsc.VectorSubcoreMesh(core_axis_name="c", subcore_axis_name="s")
    k = functools.partial(
        pl.kernel,
        mesh=mesh,
        out_type=jax.ShapeDtypeStruct((B, S, D), jnp.float32),
        scratch_types=[
            pltpu.VMEM((_CR, D), jnp.float32),
            pltpu.VMEM((_CR, D), jnp.float32),
            pltpu.VMEM((_CR, D), jnp.float32),
            pltpu.VMEM((_CR, D), jnp.float32),
            pltpu.SemaphoreType.DMA,
            pltpu.SemaphoreType.DMA,
            pltpu.SemaphoreType.DMA,
            pltpu.SemaphoreType.DMA,
            pltpu.SemaphoreType.DMA,
            pltpu.SemaphoreType.DMA,
        ],
    )(_sc_body)
    return k(X, pos_table[:S])


kernel = _kernel_tc
